# baseline (speedup 1.0000x reference)
import functools

import jax
import jax.numpy as jnp
from jax import lax
from jax.experimental import pallas as pl
from jax.experimental.pallas import tpu as pltpu
from jax.experimental.pallas import tpu_sc as plsc

T = 8192
D = 2048
E = 16

NC = 2
NS = 16
NW = NC * NS
TW = T // NW
C = 16
NCHUNK = TW // C
RING = 48
LD = 24
NLOAD = (TW + LD - 1) // LD

_LOADS = [(k * LD, min(TW, k * LD + LD)) for k in range(NLOAD)]
_SCATS = [(j * C, j * C + C) for j in range(NCHUNK)]


def _load_deps(j):
    lo, hi = _SCATS[j]
    return [k for k, (a, b) in enumerate(_LOADS) if a < hi and b > lo]


def _scat_deps(k):
    if k < 2:
        return []
    a, b = _LOADS[k]
    pa, _ = _LOADS[k - 2]
    n = b - a
    return [j for j, (x, y) in enumerate(_SCATS) if x < pa + n and y > pa]


def _dispatch_body(th_hbm, eidx_hbm, sidx_hbm, off_hbm, out_hbm,
                   eidx_v, sidx_v, off_v, buf_v, *sems):
    in_sems = sems[:NLOAD]
    out_sems = sems[NLOAD:]
    wid = lax.axis_index("s") * NC + lax.axis_index("c")
    base = wid * TW

    def start_in(k):
        a, b = _LOADS[k]
        return pltpu.async_copy(
            th_hbm.at[pl.ds(base + a, b - a), :],
            buf_v.at[pl.ds(a % RING, b - a), :], in_sems[k])

    in_handles = [None] * NLOAD
    out_handles = [None] * NCHUNK
    in_handles[0] = start_in(0)
    in_handles[1] = start_in(1)

    pltpu.sync_copy(eidx_hbm.at[pl.ds(base, TW)], eidx_v)
    pltpu.sync_copy(sidx_hbm.at[pl.ds(base, TW)], sidx_v)
    pltpu.sync_copy(off_hbm.at[pl.ds(0, E)], off_v)
    offs = off_v[...]

    waited_in = set()
    waited_out = set()
    next_load = 2
    for j in range(NCHUNK):
        for k in _load_deps(j):
            if k not in waited_in:
                in_handles[k].wait()
                waited_in.add(k)
        e = eidx_v[pl.ds(j * C, C)]
        s = sidx_v[pl.ds(j * C, C)]
        rows = offs.at[e].get(mode="promise_in_bounds") + s
        out_handles[j] = pltpu.async_copy(
            buf_v.at[pl.ds((j * C) % RING, C), :], out_hbm.at[rows],
            out_sems[j])
        while next_load < NLOAD and max(_scat_deps(next_load)) <= j:
            for jj in _scat_deps(next_load):
                if jj not in waited_out:
                    out_handles[jj].wait()
                    waited_out.add(jj)
            in_handles[next_load] = start_in(next_load)
            next_load += 1

    for j in range(NCHUNK):
        if j not in waited_out:
            out_handles[j].wait()


@jax.jit
def _dispatch(token_hidden, expert_idx, slot_idx, expert_offsets):
    mesh = plsc.VectorSubcoreMesh(core_axis_name="c", subcore_axis_name="s",
                                  num_cores=NC, num_subcores=NS)
    f = pl.kernel(
        _dispatch_body,
        out_type=jax.ShapeDtypeStruct((T, D), jnp.float32),
        mesh=mesh,
        scratch_types=[
            pltpu.VMEM((TW,), jnp.int32),
            pltpu.VMEM((TW,), jnp.int32),
            pltpu.VMEM((E,), jnp.int32),
            pltpu.VMEM((RING, D), jnp.float32),
            *([pltpu.SemaphoreType.DMA] * (NLOAD + NCHUNK)),
        ],
    )
    return f(token_hidden, expert_idx, slot_idx, expert_offsets)


def kernel(token_hidden, expert_idx, slot_idx, expert_offsets):
    return _dispatch(token_hidden,
                     expert_idx.astype(jnp.int32),
                     slot_idx.astype(jnp.int32),
                     expert_offsets.astype(jnp.int32))


# speedup vs baseline: 1.0167x; 1.0167x over previous
import jax
import jax.numpy as jnp
from jax import lax
from jax.experimental import pallas as pl
from jax.experimental.pallas import tpu as pltpu
from jax.experimental.pallas import tpu_sc as plsc

T = 8192
D = 2048
E = 16

NC = 2
NS = 16
NW = NC * NS
TW = T // NW
C = 16
NCHUNK = TW // C
NBUF = 3
DELAY = 0


def _dispatch_body(th_hbm, eidx_hbm, sidx_hbm, off_hbm, out_hbm,
                   eidx_v, sidx_v, off_v, buf_v, *sems):
    in_sems = sems[:NBUF]
    out_sems = sems[NBUF:]
    wid = lax.axis_index("s") * NC + lax.axis_index("c")
    base = wid * TW

    def start_in(j, slot):
        return pltpu.async_copy(
            th_hbm.at[pl.ds(base + j * C, C), :], buf_v.at[slot],
            in_sems[slot])

    in_handles = [None] * NBUF
    out_handles = [None] * NBUF
    for j in range(min(NBUF, NCHUNK)):
        in_handles[j] = start_in(j, j)

    pltpu.sync_copy(eidx_hbm.at[pl.ds(base, TW)], eidx_v)
    pltpu.sync_copy(sidx_hbm.at[pl.ds(base, TW)], sidx_v)
    pltpu.sync_copy(off_hbm.at[pl.ds(0, E)], off_v)
    offs = off_v[...]

    for j in range(NCHUNK):
        slot = j % NBUF
        e = eidx_v[pl.ds(j * C, C)]
        s = sidx_v[pl.ds(j * C, C)]
        rows = offs.at[e].get(mode="promise_in_bounds") + s
        in_handles[slot].wait()
        out_handles[slot] = pltpu.async_copy(
            buf_v.at[slot], out_hbm.at[rows], out_sems[slot])
        pj = j - DELAY
        nj = pj + NBUF
        if pj >= 0 and nj < NCHUNK:
            out_handles[pj % NBUF].wait()
            in_handles[nj % NBUF] = start_in(nj, nj % NBUF)

    for j in range(max(0, NCHUNK - NBUF), NCHUNK):
        out_handles[j % NBUF].wait()


@jax.jit
def _dispatch(token_hidden, expert_idx, slot_idx, expert_offsets):
    mesh = plsc.VectorSubcoreMesh(core_axis_name="c", subcore_axis_name="s",
                                  num_cores=NC, num_subcores=NS)
    f = pl.kernel(
        _dispatch_body,
        out_type=jax.ShapeDtypeStruct((T, D), jnp.float32),
        mesh=mesh,
        scratch_types=[
            pltpu.VMEM((TW,), jnp.int32),
            pltpu.VMEM((TW,), jnp.int32),
            pltpu.VMEM((E,), jnp.int32),
            pltpu.VMEM((NBUF, C, D), jnp.float32),
            *([pltpu.SemaphoreType.DMA] * (2 * NBUF)),
        ],
    )
    return f(token_hidden, expert_idx, slot_idx, expert_offsets)


def kernel(token_hidden, expert_idx, slot_idx, expert_offsets):
    return _dispatch(token_hidden,
                     expert_idx.astype(jnp.int32),
                     slot_idx.astype(jnp.int32),
                     expert_offsets.astype(jnp.int32))
